# Initial kernel scaffold; baseline (speedup 1.0000x reference)
import jax
import jax.numpy as jnp
from jax.experimental import pallas as pl


def kernel(input_ids, pos_ids, word_table, pos_table):
    raise NotImplementedError("write your pallas kernel here")


# speedup vs baseline: 4.5570x; 4.5570x over previous
import functools

import jax
import jax.numpy as jnp
from jax import lax
from jax.experimental import pallas as pl
from jax.experimental.pallas import tpu as pltpu
from jax.experimental.pallas import tpu_sc as plsc

D = 64
G = 128
NC = 2
NS = 16
NW = NC * NS


def _build(B):
    npg = B // (NW * G)
    mesh = plsc.VectorSubcoreMesh(
        core_axis_name="c", subcore_axis_name="s", num_cores=NC, num_subcores=NS
    )

    @functools.partial(
        pl.kernel,
        mesh=mesh,
        out_type=jax.ShapeDtypeStruct((B, D), jnp.float32),
        scratch_types=[
            pltpu.VMEM((npg, G), jnp.int32),
            pltpu.VMEM((npg, G), jnp.int32),
            pltpu.VMEM((G, D), jnp.float32),
            pltpu.SemaphoreType.DMA,
        ],
        compiler_params=pltpu.CompilerParams(use_tc_tiling_on_sc=False),
    )
    def emb(ids_hbm, pids_hbm, word_hbm, pos_hbm, out_hbm, idxw, idxp, rows, sem):
        wid = lax.axis_index("s") * NC + lax.axis_index("c")
        pltpu.sync_copy(ids_hbm.at[pl.ds(wid * npg, npg)], idxw)
        pltpu.sync_copy(pids_hbm.at[pl.ds(wid * npg, npg)], idxp)
        base = wid * npg * G

        def body(j, carry):
            pltpu.async_copy(pos_hbm.at[idxp.at[j]], rows, sem).wait()
            pltpu.async_copy(word_hbm.at[idxw.at[j]], rows, sem, add=True).wait()
            pltpu.sync_copy(rows, out_hbm.at[pl.ds(base + j * G, G)])
            return carry

        lax.fori_loop(0, npg, body, 0)

    return emb


def kernel(input_ids, pos_ids, word_table, pos_table):
    batch, seq_len = input_ids.shape
    B = batch * seq_len
    ids = input_ids.reshape(B // G, G).astype(jnp.int32)
    pids = pos_ids.reshape(B // G, G).astype(jnp.int32)
    out = _build(B)(ids, pids, word_table, pos_table)
    return out.reshape(batch, seq_len, D)


# speedup vs baseline: 7.7371x; 1.6978x over previous
import functools

import jax
import jax.numpy as jnp
from jax import lax
from jax.experimental import pallas as pl
from jax.experimental.pallas import tpu as pltpu
from jax.experimental.pallas import tpu_sc as plsc

D = 64
MAXLEN = 200
G = 128
NBUF = 4
HALF = 2
NC = 2
NS = 16
NW = NC * NS


def _build(B):
    npg = B // (NW * G)
    gpi = NBUF * HALF
    mesh = plsc.VectorSubcoreMesh(
        core_axis_name="c", subcore_axis_name="s", num_cores=NC, num_subcores=NS
    )

    @functools.partial(
        pl.kernel,
        mesh=mesh,
        out_type=jax.ShapeDtypeStruct((B, D), jnp.float32),
        scratch_types=[
            pltpu.VMEM((npg, G), jnp.int32),
            pltpu.VMEM((npg, G), jnp.int32),
            pltpu.VMEM((gpi, G, D), jnp.float32),
            pltpu.VMEM_SHARED((MAXLEN, D), jnp.float32),
            pltpu.SemaphoreType.DMA,
            pltpu.SemaphoreType.DMA,
            pltpu.SemaphoreType.DMA,
            pltpu.SemaphoreType.DMA,
        ],
        compiler_params=pltpu.CompilerParams(use_tc_tiling_on_sc=False),
    )
    def emb(ids_hbm, pids_hbm, word_hbm, pos_hbm, out_hbm,
            idxw, idxp, rows, pos_sh, semp, semw, semo0, semo1):
        semo = (semo0, semo1)
        wid = lax.axis_index("s") * NC + lax.axis_index("c")

        @pl.when(lax.axis_index("s") == 0)
        def _():
            pltpu.sync_copy(pos_hbm, pos_sh)

        pltpu.sync_copy(ids_hbm.at[pl.ds(wid * npg, npg)], idxw)
        pltpu.sync_copy(pids_hbm.at[pl.ds(wid * npg, npg)], idxp)
        plsc.subcore_barrier()
        base = wid * npg

        def body(jj, carry):
            for h in range(HALF):
                s0 = h * NBUF
                j0 = jj * gpi + h * NBUF

                @pl.when(jj > 0)
                def _():
                    for s in range(NBUF):
                        pltpu.make_async_copy(
                            rows.at[s0 + s], out_hbm.at[pl.ds(0, G)], semo[h]
                        ).wait()

                pcs = [
                    pltpu.async_copy(
                        pos_sh.at[idxp.at[j0 + s]], rows.at[s0 + s], semp
                    )
                    for s in range(NBUF)
                ]
                for s in range(NBUF):
                    pcs[s].wait()
                wcs = [
                    pltpu.async_copy(
                        word_hbm.at[idxw.at[j0 + s]], rows.at[s0 + s], semw,
                        add=True,
                    )
                    for s in range(NBUF)
                ]
                for s in range(NBUF):
                    wcs[s].wait()
                for s in range(NBUF):
                    pltpu.async_copy(
                        rows.at[s0 + s],
                        out_hbm.at[pl.ds((base + j0 + s) * G, G)],
                        semo[h],
                    )
            return carry

        lax.fori_loop(0, npg // gpi, body, 0)
        for h in range(HALF):
            for s in range(NBUF):
                pltpu.make_async_copy(
                    rows.at[h * NBUF + s], out_hbm.at[pl.ds(0, G)], semo[h]
                ).wait()

    return emb


def kernel(input_ids, pos_ids, word_table, pos_table):
    batch, seq_len = input_ids.shape
    B = batch * seq_len
    ids = input_ids.reshape(B // G, G).astype(jnp.int32)
    pids = pos_ids.reshape(B // G, G).astype(jnp.int32)
    out = _build(B)(ids, pids, word_table, pos_table)
    return out.reshape(batch, seq_len, D)
